# Initial kernel scaffold; baseline (speedup 1.0000x reference)
import jax
import jax.numpy as jnp
from jax.experimental import pallas as pl


def kernel(x, w1, w2, router_w):
    raise NotImplementedError("write your pallas kernel here")


# speedup vs baseline: 2.1322x; 2.1322x over previous
import jax
import jax.numpy as jnp
from jax import lax
from jax.experimental import pallas as pl
from jax.experimental.pallas import tpu as pltpu

E = 8
K = 2
TB = 256


def _routing_weights(logits):
    T = logits.shape[0]
    iota_e = lax.broadcasted_iota(jnp.int32, (T, E), 1)
    m1 = jnp.max(logits, axis=1, keepdims=True)
    i1 = jnp.min(jnp.where(logits == m1, iota_e, E), axis=1, keepdims=True)
    l2 = jnp.where(iota_e == i1, -jnp.inf, logits)
    m2 = jnp.max(l2, axis=1, keepdims=True)
    i2 = jnp.min(jnp.where(l2 == m2, iota_e, E), axis=1, keepdims=True)
    p1 = 1.0 / (1.0 + jnp.exp(m2 - m1))
    p2 = 1.0 - p1
    return jnp.where(iota_e == i1, p1, 0.0) + jnp.where(iota_e == i2, p2, 0.0)


def _gelu_exact(v):
    return 0.5 * v * (1.0 + lax.erf(v * 0.7071067811865476))


def _moe_body(x_ref, rwt_ref, w1_ref, w2_ref, out_ref, wfull_scr):
    e = pl.program_id(0)
    i = pl.program_id(1)

    @pl.when((e == 0) & (i == 0))
    def _():
        logits = jnp.dot(x_ref[...], rwt_ref[...], preferred_element_type=jnp.float32)
        wfull_scr[...] = _routing_weights(logits)

    sl = pl.ds(i * TB, TB)
    h = jnp.dot(x_ref[sl, :], w1_ref[...], preferred_element_type=jnp.float32)
    h = _gelu_exact(h)
    y = jnp.dot(h, w2_ref[...], preferred_element_type=jnp.float32)
    esel = (lax.broadcasted_iota(jnp.int32, (1, E), 1) == e).astype(jnp.float32)
    wsel = jnp.sum(wfull_scr[sl, :] * esel, axis=1, keepdims=True)
    val = y * wsel

    @pl.when(e == 0)
    def _():
        out_ref[sl, :] = val

    @pl.when(e > 0)
    def _():
        out_ref[sl, :] += val


def kernel(x, w1, w2, router_w):
    Bb, Ss, Dd = x.shape
    T = Bb * Ss
    S = w1.shape[1] // E
    xf = x.reshape(T, Dd)
    rwt = router_w.T

    out = pl.pallas_call(
        _moe_body,
        grid=(E, T // TB),
        in_specs=[
            pl.BlockSpec((T, Dd), lambda e, i: (0, 0)),
            pl.BlockSpec((Dd, E), lambda e, i: (0, 0)),
            pl.BlockSpec((Dd, S), lambda e, i: (0, e)),
            pl.BlockSpec((S, Dd), lambda e, i: (e, 0)),
        ],
        out_specs=pl.BlockSpec((T, Dd), lambda e, i: (0, 0)),
        out_shape=jax.ShapeDtypeStruct((T, Dd), jnp.float32),
        scratch_shapes=[pltpu.VMEM((T, E), jnp.float32)],
    )(xf, rwt, w1, w2)
    return out.reshape(Bb, Ss, Dd)
